# baseline (speedup 1.0000x reference)
import jax
import jax.numpy as jnp
from jax import lax
from jax.experimental import pallas as pl
from jax.experimental.pallas import tpu as pltpu
from jax.experimental.pallas import tpu_sc as plsc

H = 64
NT = 10000
ND = 2000
NL = 50000
E_UR, E_UD, E_UL, E_CC = 160000, 50000, 400000, 800000

NC, NS = 2, 16
NW = NC * NS

OFF_RET, OFF_DRAM, OFF_LNK = 0, NT, NT + ND
SALL = 22016
CNTP = 10048
MSL = SALL // NS

CEDGE = 2048
CK = 128

HALF = NL // NC
ACC6 = 25088
TRASH = HALF
ACC4 = 10240

F32 = jnp.float32
I32 = jnp.int32

NEG = -1.0e38
POS = 1.0e38


_SC_PARAMS = pltpu.CompilerParams(needs_layout_passes=False,
                                  use_tc_tiling_on_sc=False)


def _sc_mesh():
    return plsc.VectorSubcoreMesh(
        core_axis_name="c", subcore_axis_name="s",
        num_cores=NC, num_subcores=NS)


def _edge_chunk_loop(w, nworkers, E, body_fn):
    nch = -(-E // CEDGE)
    mx = -(-nch // nworkers)

    def outer(i, carry):
        k = w + nworkers * i

        @pl.when(k < nch)
        def _():
            base0 = k * CEDGE
            base = jnp.minimum(base0, E - CEDGE)
            body_fn(base, base0 - base)
        return carry

    lax.fori_loop(0, mx, outer, 0)


def _fill_loop(ref, n, value):
    v = jnp.full((16,), value, F32)

    def fb(i, carry):
        ref[pl.ds(i * 16, 16)] = v
        return carry

    lax.fori_loop(0, n // 16, fb, 0)


def _g16(x, i):
    return x.at[i].get(mode="promise_in_bounds")


def _sorted_runs(key16):
    iota = lax.iota(I32, 16)
    k, perm = plsc.sort_key_val(key16, iota)
    sames = []
    for sh in (1, 2, 4, 8):
        src = jnp.maximum(iota - sh, 0)
        kg = _g16(k, src)
        sames.append(((kg == k) & (iota >= sh), src))
    kn = _g16(k, jnp.minimum(iota + 1, 15))
    last = (iota == 15) | (kn != k)
    return k, perm, sames, last


def _seg_combine(v, sames, op):
    for same, src in sames:
        v = jnp.where(same, op(v, _g16(v, src)), v)
    return v


def _k1_body(d_ur, d_ud, d_ul, f_ur, f_ud, f_ul, ssum_out, cnt_out,
             acc, cacc, idxb, valb):
    c = lax.axis_index("c")
    s = lax.axis_index("s")
    w = s * NC + c
    _fill_loop(acc, SALL, 0.0)
    _fill_loop(cacc, CNTP, 0.0)

    ones = jnp.ones((16,), F32)

    def scatter_pass(eidx_d, feat, E, off, with_cnt):
        def chunk(base, start):
            pltpu.sync_copy(eidx_d.at[pl.ds(base, CEDGE)], idxb)
            pltpu.sync_copy(feat.at[pl.ds(base, CEDGE)], valb)

            def eb(j, carry):
                sl = pl.ds(j * 16, 16)
                k, perm, sames, last = _sorted_runs(idxb[sl])
                vf = _seg_combine(_g16(valb[sl], perm), sames,
                                  lambda a, b: a + b)
                ka = k + off
                old = plsc.load_gather(acc, [ka])
                plsc.store_scatter(acc, [ka], old + vf, mask=last)
                if with_cnt:
                    vc = _seg_combine(ones, sames, lambda a, b: a + b)
                    oldc = plsc.load_gather(cacc, [k])
                    plsc.store_scatter(cacc, [k], oldc + vc, mask=last)
                return carry

            lax.fori_loop(start // 16, CEDGE // 16, eb, 0)

        _edge_chunk_loop(w, NW, E, chunk)

    scatter_pass(d_ur, f_ur, E_UR, OFF_RET, False)
    scatter_pass(d_ud, f_ud, E_UD, OFF_DRAM, False)
    scatter_pass(d_ul, f_ul, E_UL, OFF_LNK, True)
    pltpu.sync_copy(acc, ssum_out.at[pl.ds(w * SALL, SALL)])
    pltpu.sync_copy(cacc, cnt_out.at[pl.ds(w * CNTP, CNTP)])


def _k1(d_ur, d_ud, d_ul, f_ur, f_ud, f_ul):
    return pl.kernel(
        _k1_body,
        out_type=(jax.ShapeDtypeStruct((NW * SALL,), F32),
                  jax.ShapeDtypeStruct((NW * CNTP,), F32)),
        mesh=_sc_mesh(),
        compiler_params=_SC_PARAMS,
        scratch_types=[pltpu.VMEM((SALL,), F32), pltpu.VMEM((CNTP,), F32),
                       pltpu.VMEM((CEDGE,), I32), pltpu.VMEM((CEDGE,), F32)],
    )(d_ur, d_ud, d_ul, f_ur, f_ud, f_ul)


def _k2_body(s_ur, d_ur, s_ud, d_ud, s_ul, d_ul, ssum_parts,
             smax_out, smin_out,
             sall, smax, smin, tmp, tmp2, srcb, dstb, shared):
    c = lax.axis_index("c")
    s = lax.axis_index("s")
    w = s * NC + c

    base = s * MSL
    _fill_loop(tmp, MSL, 0.0)
    for r in range(NW):
        pltpu.sync_copy(ssum_parts.at[pl.ds(r * SALL + base, MSL)], tmp2)

        def ab(i, carry):
            sl = pl.ds(i * 16, 16)
            tmp[sl] = tmp[sl] + tmp2[sl]
            return carry

        lax.fori_loop(0, MSL // 16, ab, 0)
    pltpu.sync_copy(tmp, shared.at[pl.ds(base, MSL)])
    plsc.subcore_barrier()
    pltpu.sync_copy(shared, sall)

    _fill_loop(smax, SALL, NEG)
    _fill_loop(smin, SALL, POS)

    def mm_pass(eidx_s, eidx_d, E, offs, offd):
        def chunk(cbase, start):
            pltpu.sync_copy(eidx_s.at[pl.ds(cbase, CEDGE)], srcb)
            pltpu.sync_copy(eidx_d.at[pl.ds(cbase, CEDGE)], dstb)

            def eb(j, carry):
                sl = pl.ds(j * 16, 16)
                sv = plsc.load_gather(sall, [dstb[sl] + offd])
                k, perm, sames, last = _sorted_runs(srcb[sl])
                v = _g16(sv, perm)
                vmax = _seg_combine(v, sames, jnp.maximum)
                vmin = _seg_combine(v, sames, jnp.minimum)
                kk = k + offs
                om = plsc.load_gather(smax, [kk])
                plsc.store_scatter(smax, [kk], jnp.maximum(om, vmax),
                                   mask=last)
                on = plsc.load_gather(smin, [kk])
                plsc.store_scatter(smin, [kk], jnp.minimum(on, vmin),
                                   mask=last)
                return carry

            lax.fori_loop(start // 16, CEDGE // 16, eb, 0)

        _edge_chunk_loop(w, NW, E, chunk)

    mm_pass(s_ur, d_ur, E_UR, OFF_RET, OFF_RET)
    mm_pass(s_ud, d_ud, E_UD, OFF_DRAM, OFF_DRAM)
    mm_pass(s_ul, d_ul, E_UL, OFF_LNK, OFF_LNK)
    pltpu.sync_copy(smax, smax_out.at[pl.ds(w * SALL, SALL)])
    pltpu.sync_copy(smin, smin_out.at[pl.ds(w * SALL, SALL)])


def _k2(s_ur, d_ur, s_ud, d_ud, s_ul, d_ul, ssum_parts):
    return pl.kernel(
        _k2_body,
        out_type=(jax.ShapeDtypeStruct((NW * SALL,), F32),
                  jax.ShapeDtypeStruct((NW * SALL,), F32)),
        mesh=_sc_mesh(),
        compiler_params=_SC_PARAMS,
        scratch_types=[pltpu.VMEM((SALL,), F32), pltpu.VMEM((SALL,), F32),
                       pltpu.VMEM((SALL,), F32), pltpu.VMEM((MSL,), F32),
                       pltpu.VMEM((MSL,), F32),
                       pltpu.VMEM((CEDGE,), I32), pltpu.VMEM((CEDGE,), I32),
                       pltpu.VMEM_SHARED((SALL,), F32)],
    )(s_ur, d_ur, s_ud, d_ud, s_ul, d_ul, ssum_parts)


def _k3_body(smax_ref, smin_ref, wr, br, wd, bd, wl, bl, wt, bt, out):
    smax = jnp.max(smax_ref[...], axis=1, keepdims=True)
    smin = jnp.min(smin_ref[...], axis=1, keepdims=True)

    def h_rel(off, n, w, b):
        sm = smax[off:off + n, :]
        sn = smin[off:off + n, :]
        has = sm > -1.0e37
        sel = jnp.where(w > 0, sm, sn)
        return jnp.where(has, jnp.tanh(sel * w + b), 0.0)

    h_ret = h_rel(OFF_RET, NT, wr[...], br[...])
    h_dram = jnp.concatenate(
        [h_rel(OFF_DRAM, ND, wd[...], bd[...]),
         jnp.zeros((NT - ND, H), F32)], axis=0)
    h_lnk = h_rel(OFF_LNK, NT, wl[...], bl[...])
    x = jnp.concatenate([h_ret, h_dram, h_lnk], axis=1)
    out[...] = jnp.tanh(
        jnp.dot(x, wt[...], preferred_element_type=F32) + bt[...])


def _k3(smax_t, smin_t, wr, br, wd, bd, wl, bl, wt, bt):
    return pl.pallas_call(
        _k3_body,
        out_shape=jax.ShapeDtypeStruct((NT, H), F32),
    )(smax_t, smin_t, wr, br, wd, bd, wl, bl, wt, bt)


def _k4_body(eul_s, eul_d, h_task, out, gidx, sidx, rows, zb, sem, accsh):
    c = lax.axis_index("c")
    s = lax.axis_index("s")
    w = s * NC + c

    def zr(i, carry):
        for jc in (0, 16, 32, 48):
            zb[i, pl.ds(jc, 16)] = jnp.zeros((16,), F32)
        return carry

    lax.fori_loop(0, CK, zr, 0)
    for j in range(5):
        pltpu.sync_copy(zb, accsh.at[pl.ds(s * 640 + j * CK, CK)])
    plsc.subcore_barrier()

    nch = E_UL // CK
    mx = -(-nch // NW)

    def outer(i, carry):
        k = w + NW * i

        @pl.when(k < nch)
        def _():
            base = k * CK
            pltpu.sync_copy(eul_s.at[pl.ds(base, CK)], gidx)
            pltpu.sync_copy(eul_d.at[pl.ds(base, CK)], sidx)
            pltpu.async_copy(h_task.at[gidx], rows, sem).wait()
            pltpu.sync_copy(rows, accsh.at[sidx], add=True)
        return carry

    lax.fori_loop(0, mx, outer, 0)
    plsc.subcore_barrier()

    @pl.when(s < 15)
    def _():
        st = s * 632
        pltpu.sync_copy(accsh.at[pl.ds(st, 632)], out.at[c, pl.ds(st, 632)])

    @pl.when(s == 15)
    def _():
        pltpu.sync_copy(accsh.at[pl.ds(9480, 520)], out.at[c, pl.ds(9480, 520)])


def _k4(eul_s, eul_d, h_task):
    return pl.kernel(
        _k4_body,
        out_type=jax.ShapeDtypeStruct((NC, NT, H), F32),
        mesh=_sc_mesh(),
        compiler_params=_SC_PARAMS,
        scratch_types=[pltpu.VMEM((CK,), I32), pltpu.VMEM((CK,), I32),
                       pltpu.VMEM((CK, H), F32), pltpu.VMEM((CK, H), F32),
                       pltpu.SemaphoreType.DMA,
                       pltpu.VMEM_SHARED((ACC4, H), F32)],
    )(eul_s, eul_d, h_task)


def _k5_body(m0, m1, cnt_ref, out):
    msum = m0[...] + m1[...]
    cnt = jnp.sum(cnt_ref[...], axis=1, keepdims=True)
    hm = msum / jnp.maximum(cnt[:NT, :], 1.0)
    out[...] = jnp.concatenate([hm, jnp.zeros((NL - NT, H), F32)], axis=0)


def _k5(m0, m1, cnt_t):
    return pl.pallas_call(
        _k5_body,
        out_shape=jax.ShapeDtypeStruct((NL, H), F32),
    )(m0, m1, cnt_t)


def _k6_body(ecc_s, ecc_d, feat, hl, outf, outb,
             gidx0, gidx1, gidx2, sidx0, sidx1, sidx2b, fb0, fb1, fb2,
             sidx2a, sidx2c, rows0, rows1, zb,
             semi0, semi1, semi2, semg0, semg1, semsc0, semsc1, accsh):
    sidx2 = (sidx2a, sidx2c)
    semsc = (semsc0, semsc1)
    gidx = (gidx0, gidx1, gidx2)
    sidxb = (sidx0, sidx1, sidx2b)
    fbuf = (fb0, fb1, fb2)
    semi = (semi0, semi1, semi2)
    rows = (rows0, rows1)
    semg = (semg0, semg1)
    c = lax.axis_index("c")
    s = lax.axis_index("s")
    base_node = c * HALF

    def zr(i, carry):
        for jc in (0, 16, 32, 48):
            zb[i, pl.ds(jc, 16)] = jnp.zeros((16,), F32)
        return carry

    lax.fori_loop(0, CK, zr, 0)

    NCH = E_CC // CK
    MX = -(-NCH // NS)

    def kof(i):
        return s + NS * i

    def scan(e_src, e_dst, out_ref):
        zb_base = s * 1568
        for j in range(12):
            pltpu.sync_copy(zb, accsh.at[pl.ds(zb_base + j * CK, CK)])
        pltpu.sync_copy(zb.at[pl.ds(0, 32)],
                        accsh.at[pl.ds(zb_base + 12 * CK, 32)])
        plsc.subcore_barrier()

        def fire_idx(i, q):
            base = kof(i) * CK
            pltpu.async_copy(e_src.at[pl.ds(base, CK)], gidx[q], semi[q])
            pltpu.async_copy(e_dst.at[pl.ds(base, CK)], sidxb[q], semi[q])
            pltpu.async_copy(feat.at[pl.ds(base, CK)], fbuf[q], semi[q])

        def wait_idx(i, q):
            base = kof(i) * CK
            pltpu.make_async_copy(
                e_src.at[pl.ds(base, CK)], gidx[q], semi[q]).wait()
            pltpu.make_async_copy(
                e_dst.at[pl.ds(base, CK)], sidxb[q], semi[q]).wait()
            pltpu.make_async_copy(
                feat.at[pl.ds(base, CK)], fbuf[q], semi[q]).wait()

        def process(p, q):
            def rowmul(t, cc):
                f16 = fbuf[q][pl.ds(t * 16, 16)]
                for rl in range(16):
                    r = t * 16 + rl
                    fv = _g16(f16, jnp.full((16,), rl, I32))
                    for jc in (0, 16, 32, 48):
                        rows[p][r, pl.ds(jc, 16)] = (
                            rows[p][r, pl.ds(jc, 16)] * fv)
                return cc

            lax.fori_loop(0, CK // 16, rowmul, 0)
            for t in range(CK // 16):
                sl = pl.ds(t * 16, 16)
                d = sidxb[q][sl] - base_node
                ok = (d >= 0) & (d < HALF)
                sidx2[p][sl] = jnp.where(ok, d, TRASH)
            pltpu.async_copy(rows[p], accsh.at[sidx2[p]], semsc[p],
                             add=True)

        def body(i, p, q):
            q1 = (q + 1) % 3
            q2 = (q + 2) % 3
            ki = kof(i - 1)

            @pl.when((ki >= 0) & (ki < NCH))
            def _():
                pltpu.make_async_copy(rows[1 - p],
                                      accsh.at[sidx2[1 - p]],
                                      semsc[1 - p]).wait()

            @pl.when(kof(i + 1) < NCH)
            def _():
                wait_idx(i + 1, q1)
                pltpu.async_copy(hl.at[gidx[q1]], rows[1 - p],
                                 semg[1 - p])

            @pl.when(kof(i) < NCH)
            def _():
                pltpu.make_async_copy(hl.at[gidx[q]], rows[p],
                                      semg[p]).wait()

                @pl.when(kof(i + 2) < NCH)
                def _():
                    fire_idx(i + 2, q2)
                process(p, q)

        @pl.when(kof(0) < NCH)
        def _():
            fire_idx(0, 0)

        @pl.when(kof(1) < NCH)
        def _():
            fire_idx(1, 1)

        @pl.when(kof(0) < NCH)
        def _():
            wait_idx(0, 0)
            pltpu.async_copy(hl.at[gidx[0]], rows[0], semg[0])

        def outer(i6, carry):
            for u in range(6):
                body(6 * i6 + u, u % 2, u % 3)
            return carry

        lax.fori_loop(0, (MX + 5) // 6, outer, 0)
        plsc.subcore_barrier()

        @pl.when(s < 15)
        def _():
            st = s * 1568
            pltpu.sync_copy(accsh.at[pl.ds(st, 1568)],
                            out_ref.at[pl.ds(base_node + st, 1568)])

        @pl.when(s == 15)
        def _():
            pltpu.sync_copy(accsh.at[pl.ds(23520, 1480)],
                            out_ref.at[pl.ds(base_node + 23520, 1480)])
        plsc.subcore_barrier()

    scan(ecc_s, ecc_d, outf)
    scan(ecc_d, ecc_s, outb)


def _k6(ecc_s, ecc_d, feat, hl):
    scr = [pltpu.VMEM((CK,), I32)] * 3 \
        + [pltpu.VMEM((CK,), I32)] * 3 \
        + [pltpu.VMEM((CK,), F32)] * 3 \
        + [pltpu.VMEM((CK,), I32)] * 2 \
        + [pltpu.VMEM((CK, H), F32)] * 3 \
        + [pltpu.SemaphoreType.DMA] * 7 \
        + [pltpu.VMEM_SHARED((ACC6, H), F32)]
    return pl.kernel(
        _k6_body,
        out_type=(jax.ShapeDtypeStruct((NL, H), F32),
                  jax.ShapeDtypeStruct((NL, H), F32)),
        mesh=_sc_mesh(),
        compiler_params=_SC_PARAMS,
        scratch_types=scr,
    )(ecc_s, ecc_d, feat, hl)


RB = 5000


def _k7_body(hl_b, af_b, ab_b, w_ref, b_ref, out):
    t = (jnp.dot(af_b[...], w_ref[0:H, :], preferred_element_type=F32)
         + jnp.dot(ab_b[...], w_ref[H:2 * H, :], preferred_element_type=F32)
         + b_ref[...])
    out[...] = hl_b[...] + jnp.maximum(t, 0.0)


def _k7(hl, aggf, aggb, w, b):
    bs = lambda: pl.BlockSpec((RB, H), lambda i: (i, 0))
    return pl.pallas_call(
        _k7_body,
        grid=(NL // RB,),
        in_specs=[bs(), bs(), bs(),
                  pl.BlockSpec((2 * H, H), lambda i: (0, 0)),
                  pl.BlockSpec((1, H), lambda i: (0, 0))],
        out_specs=bs(),
        out_shape=jax.ShapeDtypeStruct((NL, H), F32),
    )(hl, aggf, aggb, w, b)


def kernel(eidx_use_reticle, eidx_use_dram, eidx_use_link, eidx_connect,
           feat_use_reticle, feat_use_dram, feat_use_link, feat_connect,
           W_ret, b_ret, W_dram, b_dram, W_lnkg, b_lnkg, W_task, b_task,
           W0, b0, W1, b1, n_task, n_reticle, n_dram, n_link):
    ur_s, ur_d = eidx_use_reticle[0], eidx_use_reticle[1]
    ud_s, ud_d = eidx_use_dram[0], eidx_use_dram[1]
    ul_s, ul_d = eidx_use_link[0], eidx_use_link[1]
    cc_s, cc_d = eidx_connect[0], eidx_connect[1]
    ssum_p, cnt_p = _k1(ur_d, ud_d, ul_d,
                        feat_use_reticle.reshape(E_UR),
                        feat_use_dram.reshape(E_UD),
                        feat_use_link.reshape(E_UL))
    smax_p, smin_p = _k2(ur_s, ur_d, ud_s, ud_d, ul_s, ul_d, ssum_p)
    h_task = _k3(smax_p.reshape(NW, SALL).T, smin_p.reshape(NW, SALL).T,
                 W_ret, b_ret.reshape(1, H), W_dram, b_dram.reshape(1, H),
                 W_lnkg, b_lnkg.reshape(1, H), W_task, b_task.reshape(1, H))
    msum_p = _k4(ul_s, ul_d, h_task)
    h_link = _k5(msum_p[0], msum_p[1], cnt_p.reshape(NW, CNTP).T)
    feat_cc = feat_connect.reshape(E_CC)
    for (w, b) in ((W0, b0), (W1, b1)):
        aggf, aggb = _k6(cc_s, cc_d, feat_cc, h_link)
        h_link = _k7(h_link, aggf, aggb, w, b.reshape(1, H))
    return h_link


# speedup vs baseline: 1.4617x; 1.4617x over previous
import jax
import jax.numpy as jnp
from jax import lax
from jax.experimental import pallas as pl
from jax.experimental.pallas import tpu as pltpu
from jax.experimental.pallas import tpu_sc as plsc

H = 64
NT = 10000
ND = 2000
NL = 50000
E_UR, E_UD, E_UL, E_CC = 160000, 50000, 400000, 800000

NC, NS = 2, 16
NW = NC * NS

OFF_RET, OFF_DRAM, OFF_LNK = 0, NT, NT + ND
SALL = 22016
CNTP = 10048
MSL = SALL // NS

CEDGE = 2048
CK = 128

HALF = NL // NC
ACC6 = 25088
TRASH = HALF
ACC4 = 10240

F32 = jnp.float32
I32 = jnp.int32

NEG = -1.0e38
POS = 1.0e38


_SC_PARAMS = pltpu.CompilerParams(needs_layout_passes=False,
                                  use_tc_tiling_on_sc=False)


def _sc_mesh():
    return plsc.VectorSubcoreMesh(
        core_axis_name="c", subcore_axis_name="s",
        num_cores=NC, num_subcores=NS)


def _edge_chunk_loop(w, nworkers, E, body_fn):
    nch = -(-E // CEDGE)
    mx = -(-nch // nworkers)

    def outer(i, carry):
        k = w + nworkers * i

        @pl.when(k < nch)
        def _():
            base0 = k * CEDGE
            base = jnp.minimum(base0, E - CEDGE)
            body_fn(base, base0 - base)
        return carry

    lax.fori_loop(0, mx, outer, 0)


def _fill_loop(ref, n, value):
    v = jnp.full((16,), value, F32)

    def fb(i, carry):
        ref[pl.ds(i * 16, 16)] = v
        return carry

    lax.fori_loop(0, n // 16, fb, 0)


def _g16(x, i):
    return x.at[i].get(mode="promise_in_bounds")


def _sorted_runs(key16):
    iota = lax.iota(I32, 16)
    k, perm = plsc.sort_key_val(key16, iota)
    sames = []
    for sh in (1, 2, 4, 8):
        src = jnp.maximum(iota - sh, 0)
        kg = _g16(k, src)
        sames.append(((kg == k) & (iota >= sh), src))
    kn = _g16(k, jnp.minimum(iota + 1, 15))
    last = (iota == 15) | (kn != k)
    return k, perm, sames, last


def _seg_combine(v, sames, op):
    for same, src in sames:
        v = jnp.where(same, op(v, _g16(v, src)), v)
    return v


def _k1_body(d_ur, d_ud, d_ul, f_ur, f_ud, f_ul, ssum_out, cnt_out,
             acc, cacc, idxb, valb):
    c = lax.axis_index("c")
    s = lax.axis_index("s")
    w = s * NC + c
    _fill_loop(acc, SALL, 0.0)
    _fill_loop(cacc, CNTP, 0.0)

    ones = jnp.ones((16,), F32)

    def scatter_pass(eidx_d, feat, E, off, with_cnt):
        def chunk(base, start):
            pltpu.sync_copy(eidx_d.at[pl.ds(base, CEDGE)], idxb)
            pltpu.sync_copy(feat.at[pl.ds(base, CEDGE)], valb)

            def eb(j, carry):
                sl = pl.ds(j * 16, 16)
                k, perm, sames, last = _sorted_runs(idxb[sl])
                vf = _seg_combine(_g16(valb[sl], perm), sames,
                                  lambda a, b: a + b)
                ka = k + off
                old = plsc.load_gather(acc, [ka])
                plsc.store_scatter(acc, [ka], old + vf, mask=last)
                if with_cnt:
                    vc = _seg_combine(ones, sames, lambda a, b: a + b)
                    oldc = plsc.load_gather(cacc, [k])
                    plsc.store_scatter(cacc, [k], oldc + vc, mask=last)
                return carry

            lax.fori_loop(start // 16, CEDGE // 16, eb, 0)

        _edge_chunk_loop(w, NW, E, chunk)

    scatter_pass(d_ur, f_ur, E_UR, OFF_RET, False)
    scatter_pass(d_ud, f_ud, E_UD, OFF_DRAM, False)
    scatter_pass(d_ul, f_ul, E_UL, OFF_LNK, True)
    pltpu.sync_copy(acc, ssum_out.at[pl.ds(w * SALL, SALL)])
    pltpu.sync_copy(cacc, cnt_out.at[pl.ds(w * CNTP, CNTP)])


def _k1(d_ur, d_ud, d_ul, f_ur, f_ud, f_ul):
    return pl.kernel(
        _k1_body,
        out_type=(jax.ShapeDtypeStruct((NW * SALL,), F32),
                  jax.ShapeDtypeStruct((NW * CNTP,), F32)),
        mesh=_sc_mesh(),
        compiler_params=_SC_PARAMS,
        scratch_types=[pltpu.VMEM((SALL,), F32), pltpu.VMEM((CNTP,), F32),
                       pltpu.VMEM((CEDGE,), I32), pltpu.VMEM((CEDGE,), F32)],
    )(d_ur, d_ud, d_ul, f_ur, f_ud, f_ul)


def _k2_body(s_ur, d_ur, s_ud, d_ud, s_ul, d_ul, ssum_parts,
             smax_out, smin_out,
             sall, smax, smin, tmp, tmp2, srcb, dstb, shared):
    c = lax.axis_index("c")
    s = lax.axis_index("s")
    w = s * NC + c

    base = s * MSL
    _fill_loop(tmp, MSL, 0.0)
    for r in range(NW):
        pltpu.sync_copy(ssum_parts.at[pl.ds(r * SALL + base, MSL)], tmp2)

        def ab(i, carry):
            sl = pl.ds(i * 16, 16)
            tmp[sl] = tmp[sl] + tmp2[sl]
            return carry

        lax.fori_loop(0, MSL // 16, ab, 0)
    pltpu.sync_copy(tmp, shared.at[pl.ds(base, MSL)])
    plsc.subcore_barrier()
    pltpu.sync_copy(shared, sall)

    _fill_loop(smax, SALL, NEG)
    _fill_loop(smin, SALL, POS)

    def mm_pass(eidx_s, eidx_d, E, offs, offd):
        def chunk(cbase, start):
            pltpu.sync_copy(eidx_s.at[pl.ds(cbase, CEDGE)], srcb)
            pltpu.sync_copy(eidx_d.at[pl.ds(cbase, CEDGE)], dstb)

            def eb(j, carry):
                sl = pl.ds(j * 16, 16)
                sv = plsc.load_gather(sall, [dstb[sl] + offd])
                k, perm, sames, last = _sorted_runs(srcb[sl])
                v = _g16(sv, perm)
                vmax = _seg_combine(v, sames, jnp.maximum)
                vmin = _seg_combine(v, sames, jnp.minimum)
                kk = k + offs
                om = plsc.load_gather(smax, [kk])
                plsc.store_scatter(smax, [kk], jnp.maximum(om, vmax),
                                   mask=last)
                on = plsc.load_gather(smin, [kk])
                plsc.store_scatter(smin, [kk], jnp.minimum(on, vmin),
                                   mask=last)
                return carry

            lax.fori_loop(start // 16, CEDGE // 16, eb, 0)

        _edge_chunk_loop(w, NW, E, chunk)

    mm_pass(s_ur, d_ur, E_UR, OFF_RET, OFF_RET)
    mm_pass(s_ud, d_ud, E_UD, OFF_DRAM, OFF_DRAM)
    mm_pass(s_ul, d_ul, E_UL, OFF_LNK, OFF_LNK)
    pltpu.sync_copy(smax, smax_out.at[pl.ds(w * SALL, SALL)])
    pltpu.sync_copy(smin, smin_out.at[pl.ds(w * SALL, SALL)])


def _k2(s_ur, d_ur, s_ud, d_ud, s_ul, d_ul, ssum_parts):
    return pl.kernel(
        _k2_body,
        out_type=(jax.ShapeDtypeStruct((NW * SALL,), F32),
                  jax.ShapeDtypeStruct((NW * SALL,), F32)),
        mesh=_sc_mesh(),
        compiler_params=_SC_PARAMS,
        scratch_types=[pltpu.VMEM((SALL,), F32), pltpu.VMEM((SALL,), F32),
                       pltpu.VMEM((SALL,), F32), pltpu.VMEM((MSL,), F32),
                       pltpu.VMEM((MSL,), F32),
                       pltpu.VMEM((CEDGE,), I32), pltpu.VMEM((CEDGE,), I32),
                       pltpu.VMEM_SHARED((SALL,), F32)],
    )(s_ur, d_ur, s_ud, d_ud, s_ul, d_ul, ssum_parts)


def _k3_body(smax_ref, smin_ref, wr, br, wd, bd, wl, bl, wt, bt, out):
    smax = jnp.max(smax_ref[...], axis=1, keepdims=True)
    smin = jnp.min(smin_ref[...], axis=1, keepdims=True)

    def h_rel(off, n, w, b):
        sm = smax[off:off + n, :]
        sn = smin[off:off + n, :]
        has = sm > -1.0e37
        sel = jnp.where(w > 0, sm, sn)
        return jnp.where(has, jnp.tanh(sel * w + b), 0.0)

    h_ret = h_rel(OFF_RET, NT, wr[...], br[...])
    h_dram = jnp.concatenate(
        [h_rel(OFF_DRAM, ND, wd[...], bd[...]),
         jnp.zeros((NT - ND, H), F32)], axis=0)
    h_lnk = h_rel(OFF_LNK, NT, wl[...], bl[...])
    x = jnp.concatenate([h_ret, h_dram, h_lnk], axis=1)
    out[...] = jnp.tanh(
        jnp.dot(x, wt[...], preferred_element_type=F32) + bt[...])


def _k3(smax_t, smin_t, wr, br, wd, bd, wl, bl, wt, bt):
    return pl.pallas_call(
        _k3_body,
        out_shape=jax.ShapeDtypeStruct((NT, H), F32),
    )(smax_t, smin_t, wr, br, wd, bd, wl, bl, wt, bt)


def _k4_body(eul_s, eul_d, h_task, out,
             gidx0, gidx1, sidxb0, sidxb1, rows0, rows1, zb,
             sem0, sem1, accsh):
    gidx = (gidx0, gidx1)
    sidxb = (sidxb0, sidxb1)
    rows = (rows0, rows1)
    sem = (sem0, sem1)
    c = lax.axis_index("c")
    s = lax.axis_index("s")
    w = s * NC + c

    def zr(i, carry):
        for jc in (0, 16, 32, 48):
            zb[i, pl.ds(jc, 16)] = jnp.zeros((16,), F32)
        return carry

    lax.fori_loop(0, CK, zr, 0)
    for j in range(5):
        pltpu.sync_copy(zb, accsh.at[pl.ds(s * 640 + j * CK, CK)])
    plsc.subcore_barrier()

    nch = E_UL // CK
    mx = -(-nch // NW)

    def kof(i):
        return w + NW * i

    def load_fire(i, p):
        base = kof(i) * CK
        pltpu.sync_copy(eul_s.at[pl.ds(base, CK)], gidx[p])
        pltpu.sync_copy(eul_d.at[pl.ds(base, CK)], sidxb[p])
        pltpu.async_copy(h_task.at[gidx[p]], rows[p], sem[p])

    def body(i, p):
        @pl.when(kof(i + 1) < nch)
        def _():
            load_fire(i + 1, 1 - p)

        @pl.when(kof(i) < nch)
        def _():
            pltpu.make_async_copy(h_task.at[gidx[p]], rows[p],
                                  sem[p]).wait()
            pltpu.sync_copy(rows[p], accsh.at[sidxb[p]], add=True)

    @pl.when(kof(0) < nch)
    def _():
        load_fire(0, 0)

    def outer(i2, carry):
        body(2 * i2, 0)
        body(2 * i2 + 1, 1)
        return carry

    lax.fori_loop(0, (mx + 1) // 2, outer, 0)
    plsc.subcore_barrier()

    @pl.when(s < 15)
    def _():
        st = s * 632
        pltpu.sync_copy(accsh.at[pl.ds(st, 632)], out.at[c, pl.ds(st, 632)])

    @pl.when(s == 15)
    def _():
        pltpu.sync_copy(accsh.at[pl.ds(9480, 520)], out.at[c, pl.ds(9480, 520)])


def _k4(eul_s, eul_d, h_task):
    return pl.kernel(
        _k4_body,
        out_type=jax.ShapeDtypeStruct((NC, NT, H), F32),
        mesh=_sc_mesh(),
        compiler_params=_SC_PARAMS,
        scratch_types=[pltpu.VMEM((CK,), I32), pltpu.VMEM((CK,), I32),
                       pltpu.VMEM((CK,), I32), pltpu.VMEM((CK,), I32),
                       pltpu.VMEM((CK, H), F32), pltpu.VMEM((CK, H), F32),
                       pltpu.VMEM((CK, H), F32),
                       pltpu.SemaphoreType.DMA, pltpu.SemaphoreType.DMA,
                       pltpu.VMEM_SHARED((ACC4, H), F32)],
    )(eul_s, eul_d, h_task)


def _k5_body(m0, m1, cnt_ref, out):
    msum = m0[...] + m1[...]
    cnt = jnp.sum(cnt_ref[...], axis=1, keepdims=True)
    hm = msum / jnp.maximum(cnt[:NT, :], 1.0)
    out[...] = jnp.concatenate([hm, jnp.zeros((NL - NT, H), F32)], axis=0)


def _k5(m0, m1, cnt_t):
    return pl.pallas_call(
        _k5_body,
        out_shape=jax.ShapeDtypeStruct((NL, H), F32),
    )(m0, m1, cnt_t)


def _k6_body(ecc_s, ecc_d, feat, hl, outf, outb,
             gidx0, gidx1, gidx2, sidx0, sidx1, sidx2b, fb0, fb1, fb2,
             sidx2a, sidx2c, rows0, rows1, zb,
             semi0, semi1, semi2, semg0, semg1, semsc0, semsc1, accsh):
    sidx2 = (sidx2a, sidx2c)
    semsc = (semsc0, semsc1)
    gidx = (gidx0, gidx1, gidx2)
    sidxb = (sidx0, sidx1, sidx2b)
    fbuf = (fb0, fb1, fb2)
    semi = (semi0, semi1, semi2)
    rows = (rows0, rows1)
    semg = (semg0, semg1)
    c = lax.axis_index("c")
    s = lax.axis_index("s")
    base_node = c * HALF

    def zr(i, carry):
        for jc in (0, 16, 32, 48):
            zb[i, pl.ds(jc, 16)] = jnp.zeros((16,), F32)
        return carry

    lax.fori_loop(0, CK, zr, 0)

    NCH = E_CC // CK
    MX = -(-NCH // NS)

    def kof(i):
        return s + NS * i

    def scan(e_src, e_dst, out_ref):
        zb_base = s * 1568
        for j in range(12):
            pltpu.sync_copy(zb, accsh.at[pl.ds(zb_base + j * CK, CK)])
        pltpu.sync_copy(zb.at[pl.ds(0, 32)],
                        accsh.at[pl.ds(zb_base + 12 * CK, 32)])
        plsc.subcore_barrier()

        def fire_idx(i, q):
            base = kof(i) * CK
            pltpu.async_copy(e_src.at[pl.ds(base, CK)], gidx[q], semi[q])
            pltpu.async_copy(e_dst.at[pl.ds(base, CK)], sidxb[q], semi[q])
            pltpu.async_copy(feat.at[pl.ds(base, CK)], fbuf[q], semi[q])

        def wait_idx(i, q):
            base = kof(i) * CK
            pltpu.make_async_copy(
                e_src.at[pl.ds(base, CK)], gidx[q], semi[q]).wait()
            pltpu.make_async_copy(
                e_dst.at[pl.ds(base, CK)], sidxb[q], semi[q]).wait()
            pltpu.make_async_copy(
                feat.at[pl.ds(base, CK)], fbuf[q], semi[q]).wait()

        def process(p, q):
            def rowmul(t, cc):
                for rl in range(16):
                    r = t * 16 + rl
                    fv = plsc.load_gather(fbuf[q], [jnp.full((16,), r, I32)])
                    for jc in (0, 16, 32, 48):
                        rows[p][r, pl.ds(jc, 16)] = (
                            rows[p][r, pl.ds(jc, 16)] * fv)
                return cc

            lax.fori_loop(0, CK // 16, rowmul, 0, unroll=2)
            for t in range(CK // 16):
                sl = pl.ds(t * 16, 16)
                d = sidxb[q][sl] - base_node
                ok = (d >= 0) & (d < HALF)
                sidx2[p][sl] = jnp.where(ok, d, TRASH)
            pltpu.async_copy(rows[p], accsh.at[sidx2[p]], semsc[p],
                             add=True)

        def body(i, p, q):
            q1 = (q + 1) % 3
            q2 = (q + 2) % 3
            ki = kof(i - 1)

            @pl.when((ki >= 0) & (ki < NCH))
            def _():
                pltpu.make_async_copy(rows[1 - p],
                                      accsh.at[sidx2[1 - p]],
                                      semsc[1 - p]).wait()

            @pl.when(kof(i + 1) < NCH)
            def _():
                wait_idx(i + 1, q1)
                pltpu.async_copy(hl.at[gidx[q1]], rows[1 - p],
                                 semg[1 - p])

            @pl.when(kof(i) < NCH)
            def _():
                pltpu.make_async_copy(hl.at[gidx[q]], rows[p],
                                      semg[p]).wait()

                @pl.when(kof(i + 2) < NCH)
                def _():
                    fire_idx(i + 2, q2)
                process(p, q)

        @pl.when(kof(0) < NCH)
        def _():
            fire_idx(0, 0)

        @pl.when(kof(1) < NCH)
        def _():
            fire_idx(1, 1)

        @pl.when(kof(0) < NCH)
        def _():
            wait_idx(0, 0)
            pltpu.async_copy(hl.at[gidx[0]], rows[0], semg[0])

        def outer(i6, carry):
            for u in range(6):
                body(6 * i6 + u, u % 2, u % 3)
            return carry

        lax.fori_loop(0, (MX + 5) // 6, outer, 0)
        plsc.subcore_barrier()

        @pl.when(s < 15)
        def _():
            st = s * 1568
            pltpu.sync_copy(accsh.at[pl.ds(st, 1568)],
                            out_ref.at[pl.ds(base_node + st, 1568)])

        @pl.when(s == 15)
        def _():
            pltpu.sync_copy(accsh.at[pl.ds(23520, 1480)],
                            out_ref.at[pl.ds(base_node + 23520, 1480)])
        plsc.subcore_barrier()

    scan(ecc_s, ecc_d, outf)
    scan(ecc_d, ecc_s, outb)


def _k6(ecc_s, ecc_d, feat, hl):
    scr = [pltpu.VMEM((CK,), I32)] * 3 \
        + [pltpu.VMEM((CK,), I32)] * 3 \
        + [pltpu.VMEM((CK,), F32)] * 3 \
        + [pltpu.VMEM((CK,), I32)] * 2 \
        + [pltpu.VMEM((CK, H), F32)] * 3 \
        + [pltpu.SemaphoreType.DMA] * 7 \
        + [pltpu.VMEM_SHARED((ACC6, H), F32)]
    return pl.kernel(
        _k6_body,
        out_type=(jax.ShapeDtypeStruct((NL, H), F32),
                  jax.ShapeDtypeStruct((NL, H), F32)),
        mesh=_sc_mesh(),
        compiler_params=_SC_PARAMS,
        scratch_types=scr,
    )(ecc_s, ecc_d, feat, hl)


RB = 5000


def _k7_body(hl_b, af_b, ab_b, w_ref, b_ref, out):
    t = (jnp.dot(af_b[...], w_ref[0:H, :], preferred_element_type=F32)
         + jnp.dot(ab_b[...], w_ref[H:2 * H, :], preferred_element_type=F32)
         + b_ref[...])
    out[...] = hl_b[...] + jnp.maximum(t, 0.0)


def _k7(hl, aggf, aggb, w, b):
    bs = lambda: pl.BlockSpec((RB, H), lambda i: (i, 0))
    return pl.pallas_call(
        _k7_body,
        grid=(NL // RB,),
        in_specs=[bs(), bs(), bs(),
                  pl.BlockSpec((2 * H, H), lambda i: (0, 0)),
                  pl.BlockSpec((1, H), lambda i: (0, 0))],
        out_specs=bs(),
        out_shape=jax.ShapeDtypeStruct((NL, H), F32),
    )(hl, aggf, aggb, w, b)


def kernel(eidx_use_reticle, eidx_use_dram, eidx_use_link, eidx_connect,
           feat_use_reticle, feat_use_dram, feat_use_link, feat_connect,
           W_ret, b_ret, W_dram, b_dram, W_lnkg, b_lnkg, W_task, b_task,
           W0, b0, W1, b1, n_task, n_reticle, n_dram, n_link):
    ur_s, ur_d = eidx_use_reticle[0], eidx_use_reticle[1]
    ud_s, ud_d = eidx_use_dram[0], eidx_use_dram[1]
    ul_s, ul_d = eidx_use_link[0], eidx_use_link[1]
    cc_s, cc_d = eidx_connect[0], eidx_connect[1]
    ssum_p, cnt_p = _k1(ur_d, ud_d, ul_d,
                        feat_use_reticle.reshape(E_UR),
                        feat_use_dram.reshape(E_UD),
                        feat_use_link.reshape(E_UL))
    smax_p, smin_p = _k2(ur_s, ur_d, ud_s, ud_d, ul_s, ul_d, ssum_p)
    h_task = _k3(smax_p.reshape(NW, SALL).T, smin_p.reshape(NW, SALL).T,
                 W_ret, b_ret.reshape(1, H), W_dram, b_dram.reshape(1, H),
                 W_lnkg, b_lnkg.reshape(1, H), W_task, b_task.reshape(1, H))
    msum_p = _k4(ul_s, ul_d, h_task)
    h_link = _k5(msum_p[0], msum_p[1], cnt_p.reshape(NW, CNTP).T)
    feat_cc = feat_connect.reshape(E_CC)
    for (w, b) in ((W0, b0), (W1, b1)):
        aggf, aggb = _k6(cc_s, cc_d, feat_cc, h_link)
        h_link = _k7(h_link, aggf, aggb, w, b.reshape(1, H))
    return h_link
